# baseline (speedup 1.0000x reference)
import functools

import jax
import jax.numpy as jnp
from jax import lax
from jax.experimental import pallas as pl
from jax.experimental.pallas import tpu as pltpu
from jax.experimental.pallas import tpu_sc as plsc

DIM = 64
B_POS = 16384
B_NEG = 81920
B_TOT = B_POS + B_NEG

NC = 2
NS = 16
NW = NC * NS
PER_W = B_TOT // NW
CH = 256
NCH = PER_W // CH
GROUPS = CH // 16


def _sc_dot_kernel(idxv_hbm, idxu_hbm, vtab, utab, out_hbm,
                   idxv_all, idxu_all,
                   va, ua, vb, ub, zbuf,
                   semva, semua, semvb, semub):
    wid = lax.axis_index("s") * NC + lax.axis_index("c")
    pltpu.sync_copy(idxv_hbm.at[wid], idxv_all)
    pltpu.sync_copy(idxu_hbm.at[wid], idxu_all)

    bufs = ((va, ua, semva, semua), (vb, ub, semvb, semub))

    def start(t):
        vB, uB, sv, su = bufs[t % 2]
        hs = []
        for h in range(2):
            dv = vB.at[pl.ds(h * 128, 128)]
            du = uB.at[pl.ds(h * 128, 128)]
            hs.append(pltpu.async_copy(vtab.at[idxv_all.at[t, h]], dv, sv))
            hs.append(pltpu.async_copy(utab.at[idxu_all.at[t, h]], du, su))
        return hs

    pending = {0: start(0)}
    for t in range(NCH):
        if t + 1 < NCH:
            pending[t + 1] = start(t + 1)
        for h in pending.pop(t):
            h.wait()
        vB, uB = bufs[t % 2][0], bufs[t % 2][1]

        def gbody(g, carry, vB=vB, uB=uB):
            lane = lax.broadcasted_iota(jnp.int32, (16,), 0)
            acc = jnp.zeros((16,), jnp.float32)
            for j in range(16):
                r = g * 16 + j
                prod = jnp.zeros((16,), jnp.float32)
                for k in range(DIM // 16):
                    a = vB[r, pl.ds(k * 16, 16)]
                    b = uB[r, pl.ds(k * 16, 16)]
                    prod = prod + a * b
                s = jnp.sum(prod)
                acc = jnp.where(lane == j, s, acc)
            zbuf[pl.ds(pl.multiple_of(g * 16, 16), 16)] = acc
            return carry

        lax.fori_loop(0, GROUPS, gbody, jnp.int32(0))
        pltpu.sync_copy(zbuf, out_hbm.at[wid, t])


def _sc_dot(idx_v, idx_u, v_table, u_table):
    mesh = plsc.VectorSubcoreMesh(core_axis_name="c", subcore_axis_name="s")
    k = functools.partial(
        pl.kernel,
        mesh=mesh,
        compiler_params=pltpu.CompilerParams(
            needs_layout_passes=False, use_tc_tiling_on_sc=False),
        out_type=jax.ShapeDtypeStruct((NW, NCH, CH), jnp.float32),
        scratch_types=[
            pltpu.VMEM((NCH, 2, 128), jnp.int32),
            pltpu.VMEM((NCH, 2, 128), jnp.int32),
            pltpu.VMEM((CH, DIM), jnp.float32),
            pltpu.VMEM((CH, DIM), jnp.float32),
            pltpu.VMEM((CH, DIM), jnp.float32),
            pltpu.VMEM((CH, DIM), jnp.float32),
            pltpu.VMEM((CH,), jnp.float32),
            pltpu.SemaphoreType.DMA,
            pltpu.SemaphoreType.DMA,
            pltpu.SemaphoreType.DMA,
            pltpu.SemaphoreType.DMA,
        ],
    )(_sc_dot_kernel)
    return k(idx_v, idx_u, v_table, u_table)


def _loss_body(z_ref, o_ref):
    z = z_ref[...]
    rows = lax.broadcasted_iota(jnp.int32, z.shape, 0)
    sign = jnp.where(rows < (B_POS // 128), 1.0, -1.0)
    x = sign * z
    a = jnp.minimum(x, 0.0) - jnp.log1p(jnp.exp(-jnp.abs(x)))
    o_ref[0, 0] = -jnp.sum(a)


def kernel(pos_v, pos_u, neg_v, neg_u, v_table, u_table):
    idx_v = jnp.concatenate([pos_v, neg_v]).astype(jnp.int32).reshape(NW, NCH, 2, 128)
    idx_u = jnp.concatenate([pos_u, neg_u]).astype(jnp.int32).reshape(NW, NCH, 2, 128)
    z = _sc_dot(idx_v, idx_u, v_table, u_table)
    z2 = z.reshape(B_TOT // 128, 128)
    loss = pl.pallas_call(
        _loss_body,
        out_shape=jax.ShapeDtypeStruct((1, 1), jnp.float32),
        out_specs=pl.BlockSpec(memory_space=pltpu.SMEM),
    )(z2)
    return loss[0, 0]
